# baseline (speedup 1.0000x reference)
import jax
import jax.numpy as jnp
from jax.experimental import pallas as pl
from jax.experimental.pallas import tpu as pltpu

_K = 100000
_B = 16384
_D = 64
_W = 2048
_MR = (_K * _D) // _W
_VR = (_B * _D) // _W
_BLKR = 256
_VB = _VR // _BLKR
_NB = pl.cdiv(_MR, _BLKR)


def _push_body(mem_ref, val_ref, out_ref):
    i = pl.program_id(0)

    @pl.when(i < _VB)
    def _():
        out_ref[...] = val_ref[...]

    @pl.when(i >= _VB)
    def _():
        out_ref[...] = mem_ref[...]


def kernel(mem, value):
    out = pl.pallas_call(
        _push_body,
        grid=(_NB,),
        in_specs=[
            pl.BlockSpec((_BLKR, _W), lambda i: (jnp.maximum(i, _VB), 0)),
            pl.BlockSpec((_BLKR, _W), lambda i: (jnp.minimum(i, _VB - 1), 0)),
        ],
        out_specs=pl.BlockSpec((_BLKR, _W), lambda i: (i, 0)),
        out_shape=jax.ShapeDtypeStruct((_MR, _W), jnp.float32),
    )(mem.reshape(_MR, _W), value.reshape(_VR, _W))
    return out.reshape(_K, _D)


# speedup vs baseline: 1.5113x; 1.5113x over previous
import jax
import jax.numpy as jnp
from jax.experimental import pallas as pl
from jax.experimental.pallas import tpu as pltpu

_K = 100000
_B = 16384
_D = 64
_BLKR = 4096
_VB = _B // _BLKR
_NB = pl.cdiv(_K, _BLKR)


def _push_body(mem_ref, val_ref, out_ref):
    i = pl.program_id(0)

    @pl.when(i < _VB)
    def _():
        out_ref[...] = val_ref[...]

    @pl.when(i >= _VB)
    def _():
        out_ref[...] = mem_ref[...]


def kernel(mem, value):
    return pl.pallas_call(
        _push_body,
        grid=(_NB,),
        in_specs=[
            pl.BlockSpec((_BLKR, _D), lambda i: (jnp.maximum(i, _VB), 0)),
            pl.BlockSpec((_BLKR, _D), lambda i: (jnp.minimum(i, _VB - 1), 0)),
        ],
        out_specs=pl.BlockSpec((_BLKR, _D), lambda i: (i, 0)),
        out_shape=jax.ShapeDtypeStruct((_K, _D), jnp.float32),
    )(mem, value)


# speedup vs baseline: 6.6183x; 4.3794x over previous
import jax
import jax.numpy as jnp
from jax.experimental import pallas as pl
from jax.experimental.pallas import tpu as pltpu

_K = 100000
_B = 16384
_D = 64
_CB = 4096
_VB = _B // _CB
_NB = pl.cdiv(_K, _CB)


def _push_body(mem_ref, val_ref, out_ref):
    i = pl.program_id(0)

    @pl.when(i < _VB)
    def _():
        out_ref[...] = val_ref[...]

    @pl.when(i >= _VB)
    def _():
        out_ref[...] = mem_ref[...]


def kernel(mem, value):
    out_t = pl.pallas_call(
        _push_body,
        grid=(_NB,),
        in_specs=[
            pl.BlockSpec((_D, _CB), lambda i: (0, jnp.maximum(i, _VB))),
            pl.BlockSpec((_D, _CB), lambda i: (0, jnp.minimum(i, _VB - 1))),
        ],
        out_specs=pl.BlockSpec((_D, _CB), lambda i: (0, i)),
        out_shape=jax.ShapeDtypeStruct((_D, _K), jnp.float32),
    )(mem.T, value.T)
    return out_t.T
